# baseline (speedup 1.0000x reference)
import jax
import jax.numpy as jnp
from jax.experimental import pallas as pl
from jax.experimental.pallas import tpu as pltpu

DIM = 1024
HIDDEN = 4096
N_EXP = 8
H_BLK = 512
N_HBLK = HIDDEN // H_BLK


def _moe_dense_kernel(x_ref, xb_ref, wg_ref, w1_ref, w2_ref, out_ref, gate_scr):
    e = pl.program_id(0)
    h = pl.program_id(1)

    @pl.when((e == 0) & (h == 0))
    def _gate():
        logits = jax.lax.dot_general(
            xb_ref[...], wg_ref[...].astype(jnp.bfloat16),
            (((1,), (1,)), ((), ())),
            preferred_element_type=jnp.float32)
        iota = jax.lax.broadcasted_iota(jnp.int32, logits.shape, 1)
        v1 = jnp.max(logits, axis=-1, keepdims=True)
        i1 = jnp.min(jnp.where(logits == v1, iota, N_EXP), axis=-1, keepdims=True)
        oh1 = iota == i1
        l2 = jnp.where(oh1, -jnp.inf, logits)
        v2 = jnp.max(l2, axis=-1, keepdims=True)
        i2 = jnp.min(jnp.where(l2 == v2, iota, N_EXP), axis=-1, keepdims=True)
        oh2 = iota == i2
        w1w = 1.0 / (1.0 + jnp.exp(v2 - v1))
        w2w = 1.0 - w1w
        gate_scr[...] = jnp.where(oh1, w1w, 0.0) + jnp.where(oh2, w2w, 0.0)
        out_ref[...] = jnp.zeros_like(out_ref)

    a = jnp.dot(xb_ref[...], w1_ref[0].T, preferred_element_type=jnp.float32)
    a = a * (1.0 / (1.0 + jnp.exp(-a)))
    y = jnp.dot(a.astype(jnp.bfloat16), w2_ref[0].T,
                preferred_element_type=jnp.float32)
    iota = jax.lax.broadcasted_iota(jnp.int32, gate_scr.shape, 1)
    w_col = jnp.sum(jnp.where(iota == e, gate_scr[...], 0.0), axis=1,
                    keepdims=True)
    out_ref[...] += w_col * y


def kernel(x, Wg, W1, W2):
    B, T, D = x.shape
    x_flat = x.reshape(T, D)
    xb = x_flat.astype(jnp.bfloat16)
    w1b = W1.astype(jnp.bfloat16)
    w2b = W2.astype(jnp.bfloat16)

    out = pl.pallas_call(
        _moe_dense_kernel,
        grid=(N_EXP, N_HBLK),
        in_specs=[
            pl.BlockSpec((T, D), lambda e, h: (0, 0)),
            pl.BlockSpec((T, D), lambda e, h: (0, 0)),
            pl.BlockSpec((N_EXP, D), lambda e, h: (0, 0)),
            pl.BlockSpec((1, H_BLK, D), lambda e, h: (e, h, 0)),
            pl.BlockSpec((1, D, H_BLK), lambda e, h: (e, 0, h)),
        ],
        out_specs=pl.BlockSpec((T, D), lambda e, h: (0, 0)),
        out_shape=jax.ShapeDtypeStruct((T, D), jnp.float32),
        scratch_shapes=[pltpu.VMEM((T, N_EXP), jnp.float32)],
        compiler_params=pltpu.CompilerParams(
            dimension_semantics=("arbitrary", "arbitrary")),
    )(x_flat, xb, Wg, w1b, w2b)
    return out.reshape(B, T, D)
